# baseline (speedup 1.0000x reference)
import functools

import jax
import jax.numpy as jnp
from jax import lax
from jax.experimental import pallas as pl
from jax.experimental.pallas import tpu as pltpu
from jax.experimental.pallas import tpu_sc as plsc

B = 16384
F = 26
V = 100000
D = 32
NNUM = 13
H1 = 512
H2 = 256
ED = F * D
ED2 = 896
EPS = 1e-5

NC = 2
NS = 16
NW = NC * NS
BPW = B // NW
KPW = BPW * F
GB = 8
CI = GB * F
NCH = BPW // GB
TROWS = F * V * D // 128

BM = 1024
NT = B // BM


@functools.lru_cache(maxsize=None)
def _make_sc_gather():
    mesh = plsc.VectorSubcoreMesh(core_axis_name="c", subcore_axis_name="s",
                                  num_cores=NC, num_subcores=NS)

    @functools.partial(
        pl.kernel,
        mesh=mesh,
        out_type=jax.ShapeDtypeStruct((B * ED2,), jnp.float32),
        scratch_types=[
            pltpu.VMEM((KPW,), jnp.int32),
            pltpu.VMEM((KPW,), jnp.int32),
            pltpu.VMEM((KPW,), jnp.int32),
            pltpu.VMEM((CI, 128), jnp.float32),
            pltpu.VMEM((CI, 128), jnp.float32),
            pltpu.VMEM((GB * ED2,), jnp.float32),
            pltpu.SemaphoreType.DMA,
            pltpu.SemaphoreType.DMA,
        ],
        compiler_params=pltpu.CompilerParams(use_tc_tiling_on_sc=True,
                                             needs_layout_passes=False),
    )
    def _sc_gather(tp_hbm, jr_hbm, wc_hbm, db_hbm, out_hbm,
                   jv, wv, dv, buf0, buf1, stg, s0, s1):
        wid = lax.axis_index("s") * NC + lax.axis_index("c")
        i0 = wid * KPW
        b0 = wid * BPW
        pltpu.sync_copy(jr_hbm.at[pl.ds(i0, KPW)], jv)
        pltpu.sync_copy(wc_hbm.at[pl.ds(i0, KPW)], wv)
        pltpu.sync_copy(db_hbm.at[pl.ds(i0, KPW)], dv)

        lanes = lax.iota(jnp.int32, 16)
        zero16 = jnp.zeros((16,), jnp.float32)

        def zpad(i, carry):
            r = i // 4
            base = r * ED2 + ED + (i % 4) * 16
            plsc.store_scatter(stg, [base + lanes], zero16)
            return carry
        lax.fori_loop(0, GB * 4, zpad, 0)

        def extract(c, buf):
            def grp(t, carry):
                k0 = c * CI + t * 16
                rows = t * 16 + lanes
                wvv = wv[pl.ds(k0, 16)]
                dvv = dv[pl.ds(k0, 16)]
                for d in range(D):
                    vals = plsc.load_gather(buf, [rows, wvv + d])
                    plsc.store_scatter(stg, [dvv + d], vals)
                return carry
            lax.fori_loop(0, CI // 16, grp, 0)
            pltpu.sync_copy(stg, out_hbm.at[pl.ds((b0 + c * GB) * ED2,
                                                  GB * ED2)])

        def gidx(c):
            return jv.at[pl.ds(c * CI, CI)]

        pltpu.async_copy(tp_hbm.at[gidx(0)], buf0, s0)

        def body(h, carry):
            c0 = 2 * h
            pltpu.async_copy(tp_hbm.at[gidx(c0 + 1)], buf1, s1)
            pltpu.make_async_copy(tp_hbm.at[gidx(c0)], buf0, s0).wait()
            extract(c0, buf0)

            @pl.when(h < NCH // 2 - 1)
            def _():
                pltpu.async_copy(tp_hbm.at[gidx(c0 + 2)], buf0, s0)

            pltpu.make_async_copy(tp_hbm.at[gidx(c0 + 1)], buf1, s1).wait()
            extract(c0 + 1, buf1)
            return carry

        lax.fori_loop(0, NCH // 2, body, 0)

    return _sc_gather


def _gather(tpack, jrow, wcol, dstb):
    return _make_sc_gather()(tpack, jrow, wcol, dstb)


def _nt_dot(a, b):
    return lax.dot_general(a, b, (((1,), (1,)), ((), ())),
                           precision=lax.Precision.HIGHEST,
                           preferred_element_type=jnp.float32)


def _k1_body(emb_ref, xc_ref, w1e_ref, w1n_ref, b1_ref, gn_ref, bn_ref,
             h1_ref, st_ref):
    i = pl.program_id(0)
    xc = xc_ref[...]
    mu = jnp.mean(xc, axis=0, keepdims=True)
    ex2 = jnp.mean(xc * xc, axis=0, keepdims=True)
    var = ex2 - mu * mu
    sc = gn_ref[...] * lax.rsqrt(var + EPS)
    sh = bn_ref[...] - mu * sc
    xt = xc_ref[pl.ds(i * BM, BM), :] * sc + sh
    z = _nt_dot(emb_ref[...], w1e_ref[...]) + _nt_dot(xt, w1n_ref[...])
    h = jnp.maximum(z + b1_ref[...], 0.0)
    h1_ref[...] = h

    @pl.when(i == 0)
    def _():
        st_ref[...] = jnp.zeros_like(st_ref)

    st_ref[0:1, :] += jnp.sum(h, axis=0, keepdims=True)
    st_ref[1:2, :] += jnp.sum(h * h, axis=0, keepdims=True)


def _k2_body(h1_ref, st1_ref, w2_ref, b2_ref, g1_ref, be1_ref, h2_ref, st_ref):
    i = pl.program_id(0)
    mu = st1_ref[0:1, :] * (1.0 / B)
    var = st1_ref[1:2, :] * (1.0 / B) - mu * mu
    sc = g1_ref[...] * lax.rsqrt(var + EPS)
    sh = be1_ref[...] - mu * sc
    hn = h1_ref[...] * sc + sh
    h = jnp.maximum(_nt_dot(hn, w2_ref[...]) + b2_ref[...], 0.0)
    h2_ref[...] = h

    @pl.when(i == 0)
    def _():
        st_ref[...] = jnp.zeros_like(st_ref)

    st_ref[0:1, :] += jnp.sum(h, axis=0, keepdims=True)
    st_ref[1:2, :] += jnp.sum(h * h, axis=0, keepdims=True)


def _k3_body(h2_ref, st2_ref, wout_ref, bout_ref, g2_ref, be2_ref, out_ref):
    mu = st2_ref[0:1, :] * (1.0 / B)
    var = st2_ref[1:2, :] * (1.0 / B) - mu * mu
    sc = g2_ref[...] * lax.rsqrt(var + EPS)
    sh = be2_ref[...] - mu * sc
    hn = h2_ref[...] * sc + sh
    z = jnp.sum(hn * wout_ref[...], axis=1, keepdims=True) + bout_ref[...]
    out_ref[...] = 1.0 / (1.0 + jnp.exp(-z))


_k1 = pl.pallas_call(
    _k1_body,
    grid=(NT,),
    in_specs=[
        pl.BlockSpec((BM, ED2), lambda i: (i, 0)),
        pl.BlockSpec((B, NNUM), lambda i: (0, 0)),
        pl.BlockSpec((H1, ED2), lambda i: (0, 0)),
        pl.BlockSpec((H1, NNUM), lambda i: (0, 0)),
        pl.BlockSpec((1, H1), lambda i: (0, 0)),
        pl.BlockSpec((1, NNUM), lambda i: (0, 0)),
        pl.BlockSpec((1, NNUM), lambda i: (0, 0)),
    ],
    out_specs=[
        pl.BlockSpec((BM, H1), lambda i: (i, 0)),
        pl.BlockSpec((8, H1), lambda i: (0, 0)),
    ],
    out_shape=[
        jax.ShapeDtypeStruct((B, H1), jnp.float32),
        jax.ShapeDtypeStruct((8, H1), jnp.float32),
    ],
)

_k2 = pl.pallas_call(
    _k2_body,
    grid=(NT,),
    in_specs=[
        pl.BlockSpec((BM, H1), lambda i: (i, 0)),
        pl.BlockSpec((8, H1), lambda i: (0, 0)),
        pl.BlockSpec((H2, H1), lambda i: (0, 0)),
        pl.BlockSpec((1, H2), lambda i: (0, 0)),
        pl.BlockSpec((1, H1), lambda i: (0, 0)),
        pl.BlockSpec((1, H1), lambda i: (0, 0)),
    ],
    out_specs=[
        pl.BlockSpec((BM, H2), lambda i: (i, 0)),
        pl.BlockSpec((8, H2), lambda i: (0, 0)),
    ],
    out_shape=[
        jax.ShapeDtypeStruct((B, H2), jnp.float32),
        jax.ShapeDtypeStruct((8, H2), jnp.float32),
    ],
)

_k3 = pl.pallas_call(
    _k3_body,
    grid=(NT,),
    in_specs=[
        pl.BlockSpec((BM, H2), lambda i: (i, 0)),
        pl.BlockSpec((8, H2), lambda i: (0, 0)),
        pl.BlockSpec((1, H2), lambda i: (0, 0)),
        pl.BlockSpec((1, 1), lambda i: (0, 0)),
        pl.BlockSpec((1, H2), lambda i: (0, 0)),
        pl.BlockSpec((1, H2), lambda i: (0, 0)),
    ],
    out_specs=pl.BlockSpec((BM, 1), lambda i: (i, 0)),
    out_shape=jax.ShapeDtypeStruct((B, 1), jnp.float32),
)


def kernel(x_cat, x_cont, tables, W1, b1, g1, be1, W2, b2, g2, be2,
           Wout, bout, gnum, bnum):
    tpack = tables.reshape(TROWS, 128)
    xc32 = x_cat.astype(jnp.int32)
    frow = (jnp.arange(F, dtype=jnp.int32) * (V * D // 128))[None, :]
    jrow = (frow + xc32 // 4).reshape(-1)
    wcol = ((xc32 % 4) * D).reshape(-1)
    kk = jnp.arange(B * F, dtype=jnp.int32)
    dstb = ((kk // F) % GB) * ED2 + (kk % F) * D

    emb = _gather(tpack, jrow, wcol, dstb).reshape(B, ED2)

    w1e = jnp.concatenate(
        [W1[:, :ED], jnp.zeros((H1, ED2 - ED), jnp.float32)], axis=1)
    w1n = W1[:, ED:]
    h1, st1 = _k1(emb, x_cont, w1e, w1n, b1.reshape(1, H1),
                  gnum.reshape(1, NNUM), bnum.reshape(1, NNUM))
    h2, st2 = _k2(h1, st1, W2, b2.reshape(1, H2),
                  g1.reshape(1, H1), be1.reshape(1, H1))
    out = _k3(h2, st2, Wout.reshape(1, H2), bout.reshape(1, 1),
              g2.reshape(1, H2), be2.reshape(1, H2))
    return out.reshape(B)


# speedup vs baseline: 1.2388x; 1.2388x over previous
import functools

import jax
import jax.numpy as jnp
from jax import lax
from jax.experimental import pallas as pl
from jax.experimental.pallas import tpu as pltpu
from jax.experimental.pallas import tpu_sc as plsc

B = 16384
F = 26
V = 100000
D = 32
NNUM = 13
H1 = 512
H2 = 256
ED = F * D
ED2 = 896
EPS = 1e-5

NC = 2
NS = 16
NW = NC * NS
BPW = B // NW
KPW = BPW * F
GB = 8
CI = GB * F
NCH = BPW // GB
TROWS = F * V * D // 128

BM = 1024
NT = B // BM


@functools.lru_cache(maxsize=None)
def _make_sc_gather():
    mesh = plsc.VectorSubcoreMesh(core_axis_name="c", subcore_axis_name="s",
                                  num_cores=NC, num_subcores=NS)

    @functools.partial(
        pl.kernel,
        mesh=mesh,
        out_type=jax.ShapeDtypeStruct((B * ED2,), jnp.float32),
        scratch_types=[
            pltpu.VMEM((KPW,), jnp.int32),
            pltpu.VMEM((KPW,), jnp.int32),
            pltpu.VMEM((KPW,), jnp.int32),
            pltpu.VMEM((CI, 128), jnp.float32),
            pltpu.VMEM((CI, 128), jnp.float32),
            pltpu.VMEM((GB * ED2,), jnp.float32),
            pltpu.SemaphoreType.DMA,
            pltpu.SemaphoreType.DMA,
        ],
        compiler_params=pltpu.CompilerParams(use_tc_tiling_on_sc=True,
                                             needs_layout_passes=False,
                                             disable_bounds_checks=True),
    )
    def _sc_gather(tp_hbm, jr_hbm, wc_hbm, db_hbm, out_hbm,
                   jv, wv, dv, buf0, buf1, stg, s0, s1):
        wid = lax.axis_index("s") * NC + lax.axis_index("c")
        i0 = wid * KPW
        b0 = wid * BPW
        pltpu.sync_copy(jr_hbm.at[pl.ds(i0, KPW)], jv)
        pltpu.sync_copy(wc_hbm.at[pl.ds(i0, KPW)], wv)
        pltpu.sync_copy(db_hbm.at[pl.ds(i0, KPW)], dv)

        lanes = lax.iota(jnp.int32, 16)
        zero16 = jnp.zeros((16,), jnp.float32)

        def zpad(i, carry):
            r = i // 4
            base = r * ED2 + ED + (i % 4) * 16
            plsc.store_scatter(stg, [base + lanes], zero16)
            return carry
        lax.fori_loop(0, GB * 4, zpad, 0)

        def extract(c, buf):
            def grp(t, carry):
                k0 = c * CI + t * 16
                wvv = wv[pl.ds(k0, 16)]
                dvv = dv[pl.ds(k0, 16)]
                for u in range(16):
                    p = t * 16 + u
                    off = wvv[u]
                    dst = dvv[u]
                    stg[pl.ds(dst, 16)] = buf[p, pl.ds(off, 16)]
                    stg[pl.ds(dst + 16, 16)] = buf[p, pl.ds(off + 16, 16)]
                return carry
            lax.fori_loop(0, CI // 16, grp, 0)
            pltpu.sync_copy(stg, out_hbm.at[pl.ds((b0 + c * GB) * ED2,
                                                  GB * ED2)])

        def gidx(c):
            return jv.at[pl.ds(c * CI, CI)]

        pltpu.async_copy(tp_hbm.at[gidx(0)], buf0, s0)

        def body(h, carry):
            c0 = 2 * h
            pltpu.async_copy(tp_hbm.at[gidx(c0 + 1)], buf1, s1)
            pltpu.make_async_copy(tp_hbm.at[gidx(c0)], buf0, s0).wait()
            extract(c0, buf0)

            @pl.when(h < NCH // 2 - 1)
            def _():
                pltpu.async_copy(tp_hbm.at[gidx(c0 + 2)], buf0, s0)

            pltpu.make_async_copy(tp_hbm.at[gidx(c0 + 1)], buf1, s1).wait()
            extract(c0 + 1, buf1)
            return carry

        lax.fori_loop(0, NCH // 2, body, 0)

    return _sc_gather


def _gather(tpack, jrow, wcol, dstb):
    return _make_sc_gather()(tpack, jrow, wcol, dstb)


def _nt_dot(a, b):
    return lax.dot_general(a, b, (((1,), (1,)), ((), ())),
                           precision=lax.Precision.HIGHEST,
                           preferred_element_type=jnp.float32)


def _k1_body(emb_ref, xc_ref, w1e_ref, w1n_ref, b1_ref, gn_ref, bn_ref,
             h1_ref, st_ref):
    i = pl.program_id(0)
    xc = xc_ref[...]
    mu = jnp.mean(xc, axis=0, keepdims=True)
    ex2 = jnp.mean(xc * xc, axis=0, keepdims=True)
    var = ex2 - mu * mu
    sc = gn_ref[...] * lax.rsqrt(var + EPS)
    sh = bn_ref[...] - mu * sc
    xt = xc_ref[pl.ds(i * BM, BM), :] * sc + sh
    z = _nt_dot(emb_ref[...], w1e_ref[...]) + _nt_dot(xt, w1n_ref[...])
    h = jnp.maximum(z + b1_ref[...], 0.0)
    h1_ref[...] = h

    @pl.when(i == 0)
    def _():
        st_ref[...] = jnp.zeros_like(st_ref)

    st_ref[0:1, :] += jnp.sum(h, axis=0, keepdims=True)
    st_ref[1:2, :] += jnp.sum(h * h, axis=0, keepdims=True)


def _k2_body(h1_ref, st1_ref, w2_ref, b2_ref, g1_ref, be1_ref, h2_ref, st_ref):
    i = pl.program_id(0)
    mu = st1_ref[0:1, :] * (1.0 / B)
    var = st1_ref[1:2, :] * (1.0 / B) - mu * mu
    sc = g1_ref[...] * lax.rsqrt(var + EPS)
    sh = be1_ref[...] - mu * sc
    hn = h1_ref[...] * sc + sh
    h = jnp.maximum(_nt_dot(hn, w2_ref[...]) + b2_ref[...], 0.0)
    h2_ref[...] = h

    @pl.when(i == 0)
    def _():
        st_ref[...] = jnp.zeros_like(st_ref)

    st_ref[0:1, :] += jnp.sum(h, axis=0, keepdims=True)
    st_ref[1:2, :] += jnp.sum(h * h, axis=0, keepdims=True)


def _k3_body(h2_ref, st2_ref, wout_ref, bout_ref, g2_ref, be2_ref, out_ref):
    mu = st2_ref[0:1, :] * (1.0 / B)
    var = st2_ref[1:2, :] * (1.0 / B) - mu * mu
    sc = g2_ref[...] * lax.rsqrt(var + EPS)
    sh = be2_ref[...] - mu * sc
    hn = h2_ref[...] * sc + sh
    z = jnp.sum(hn * wout_ref[...], axis=1, keepdims=True) + bout_ref[...]
    out_ref[...] = 1.0 / (1.0 + jnp.exp(-z))


_k1 = pl.pallas_call(
    _k1_body,
    grid=(NT,),
    in_specs=[
        pl.BlockSpec((BM, ED2), lambda i: (i, 0)),
        pl.BlockSpec((B, NNUM), lambda i: (0, 0)),
        pl.BlockSpec((H1, ED2), lambda i: (0, 0)),
        pl.BlockSpec((H1, NNUM), lambda i: (0, 0)),
        pl.BlockSpec((1, H1), lambda i: (0, 0)),
        pl.BlockSpec((1, NNUM), lambda i: (0, 0)),
        pl.BlockSpec((1, NNUM), lambda i: (0, 0)),
    ],
    out_specs=[
        pl.BlockSpec((BM, H1), lambda i: (i, 0)),
        pl.BlockSpec((8, H1), lambda i: (0, 0)),
    ],
    out_shape=[
        jax.ShapeDtypeStruct((B, H1), jnp.float32),
        jax.ShapeDtypeStruct((8, H1), jnp.float32),
    ],
)

_k2 = pl.pallas_call(
    _k2_body,
    grid=(NT,),
    in_specs=[
        pl.BlockSpec((BM, H1), lambda i: (i, 0)),
        pl.BlockSpec((8, H1), lambda i: (0, 0)),
        pl.BlockSpec((H2, H1), lambda i: (0, 0)),
        pl.BlockSpec((1, H2), lambda i: (0, 0)),
        pl.BlockSpec((1, H1), lambda i: (0, 0)),
        pl.BlockSpec((1, H1), lambda i: (0, 0)),
    ],
    out_specs=[
        pl.BlockSpec((BM, H2), lambda i: (i, 0)),
        pl.BlockSpec((8, H2), lambda i: (0, 0)),
    ],
    out_shape=[
        jax.ShapeDtypeStruct((B, H2), jnp.float32),
        jax.ShapeDtypeStruct((8, H2), jnp.float32),
    ],
)

_k3 = pl.pallas_call(
    _k3_body,
    grid=(NT,),
    in_specs=[
        pl.BlockSpec((BM, H2), lambda i: (i, 0)),
        pl.BlockSpec((8, H2), lambda i: (0, 0)),
        pl.BlockSpec((1, H2), lambda i: (0, 0)),
        pl.BlockSpec((1, 1), lambda i: (0, 0)),
        pl.BlockSpec((1, H2), lambda i: (0, 0)),
        pl.BlockSpec((1, H2), lambda i: (0, 0)),
    ],
    out_specs=pl.BlockSpec((BM, 1), lambda i: (i, 0)),
    out_shape=jax.ShapeDtypeStruct((B, 1), jnp.float32),
)


def kernel(x_cat, x_cont, tables, W1, b1, g1, be1, W2, b2, g2, be2,
           Wout, bout, gnum, bnum):
    tpack = tables.reshape(TROWS, 128)
    xc32 = x_cat.astype(jnp.int32)
    frow = (jnp.arange(F, dtype=jnp.int32) * (V * D // 128))[None, :]
    jrow = (frow + xc32 // 4).reshape(-1)
    wcol = ((xc32 % 4) * D).reshape(-1)
    kk = jnp.arange(B * F, dtype=jnp.int32)
    dstb = ((kk // F) % GB) * ED2 + (kk % F) * D

    emb = _gather(tpack, jrow, wcol, dstb).reshape(B, ED2)

    w1e = jnp.concatenate(
        [W1[:, :ED], jnp.zeros((H1, ED2 - ED), jnp.float32)], axis=1)
    w1n = W1[:, ED:]
    h1, st1 = _k1(emb, x_cont, w1e, w1n, b1.reshape(1, H1),
                  gnum.reshape(1, NNUM), bnum.reshape(1, NNUM))
    h2, st2 = _k2(h1, st1, W2, b2.reshape(1, H2),
                  g1.reshape(1, H1), be1.reshape(1, H1))
    out = _k3(h2, st2, Wout.reshape(1, H2), bout.reshape(1, 1),
              g2.reshape(1, H2), be2.reshape(1, H2))
    return out.reshape(B)
